# baseline (speedup 1.0000x reference)
import functools

import jax
import jax.numpy as jnp
from jax import lax
from jax.experimental import pallas as pl
from jax.experimental.pallas import tpu as pltpu
from jax.experimental.pallas import tpu_sc as plsc

_B = 16384
_D = 32
_CW = 1024
_U_MAIN = 999424
_I_MAIN = 99328
_U_TAIL = 999360
_I_TAIL = 99232
_NU = 1000000
_NI = 100000


@functools.lru_cache(maxsize=None)
def _make_lookup():
  info = plsc.get_sparse_core_info()
  NC, NS = info.num_cores, info.num_subcores
  mesh = plsc.VectorSubcoreMesh(core_axis_name="c", subcore_axis_name="s")

  @functools.partial(
      pl.kernel,
      mesh=mesh,
      compiler_params=pltpu.CompilerParams(needs_layout_passes=False),
      out_type=(
          jax.ShapeDtypeStruct((_B, 128), jnp.float32),
          jax.ShapeDtypeStruct((_B, 128), jnp.float32),
      ),
      scratch_types=[
          pltpu.VMEM((_B,), jnp.int32),
          pltpu.VMEM((_B,), jnp.int32),
          pltpu.VMEM((_B,), jnp.int32),
          pltpu.VMEM((_D, _CW), jnp.float32),
          pltpu.VMEM((_D, _CW), jnp.float32),
          pltpu.VMEM((64, 128), jnp.float32),
          pltpu.VMEM((4, 16), jnp.int32),
          pltpu.SemaphoreType.DMA,
          pltpu.SemaphoreType.DMA,
          pltpu.SemaphoreType.DMA,
          pltpu.SemaphoreType.DMA,
          pltpu.SemaphoreType.DMA,
          pltpu.SemaphoreType.DMA,
      ],
  )
  def lookup(uids_hbm, iids_hbm, utab_hbm, itab_hbm, utail_hbm, itail_hbm,
             uout_hbm, iout_hbm,
             ids_v, lst_v, pos_v, bufa_v, bufb_v, outring_v,
             posring_v, sma, smb, sm0, sm1, sm2, sm3):
    wid = lax.axis_index("s") * NC + lax.axis_index("c")
    lanes = lax.iota(jnp.int32, 16)
    sems = [sm0, sm1, sm2, sm3]

    def drain_scatter(sem):
      pltpu.make_async_copy(
          uout_hbm.at[pl.ds(0, 16), :],
          outring_v.at[pl.ds(0, 16), :], sem).wait()

    def run_filter(lo, hi):
      def body(g, cnt):
        u = ids_v[pl.ds(g * 16, 16)]
        m = (u >= lo) & (u < hi)
        nm = plsc.all_reduce_population_count(m)[0]

        @pl.when(nm > 0)
        def _():
          ranks = plsc.cumsum(jnp.where(m, 1, 0)) - 1
          slots = cnt + ranks
          plsc.store_scatter(lst_v, [slots], u, mask=m)
          plsc.store_scatter(pos_v, [slots], lanes + g * 16, mask=m)

        return cnt + nm
      return lax.fori_loop(0, _B // 16, body, 0)

    def process_span(buf, lane0, width, cnt, out_hbm):
      def mbody(g, mcnt):
        u = lst_v[pl.ds(g * 16, 16)]
        m = (u >= lane0) & (u < lane0 + width) & (lanes + g * 16 < cnt)
        nm = plsc.all_reduce_population_count(m)[0]

        @pl.when(nm > 0)
        def _():
          ranks = plsc.cumsum(jnp.where(m, 1, 0)) - 1
          plsc.store_scatter(ids_v, [mcnt + ranks], lanes + g * 16, mask=m)

        return mcnt + nm
      nlg = lax.div(cnt + 15, 16)
      mcnt = lax.fori_loop(0, nlg, mbody, 0)
      ngrp = lax.div(mcnt + 15, 16)

      def gbody(g4, carry):
        for s in range(4):
          g = g4 * 4 + s

          @pl.when(g < ngrp)
          def _():
            j = g * 16 + lanes
            jp = jnp.where(j < mcnt, j, jnp.full((16,), 1, jnp.int32) * g * 16)
            idxv = plsc.load_gather(ids_v, [jp])
            uv = plsc.load_gather(lst_v, [idxv])
            pv = plsc.load_gather(pos_v, [idxv])
            ll = uv - lane0

            @pl.when(g >= 4)
            def _():
              drain_scatter(sems[s])

            rows = s * 16 + lanes
            for d in range(_D):
              dv = jnp.full((16,), d, jnp.int32)
              vals = plsc.load_gather(buf, [dv, ll])
              plsc.store_scatter(outring_v, [rows, dv], vals)
            posring_v[s] = pv
            pltpu.async_copy(
                outring_v.at[pl.ds(s * 16, 16), :],
                out_hbm.at[posring_v.at[s]], sems[s])
        return carry

      lax.fori_loop(0, lax.div(ngrp + 3, 4), gbody, 0)
      for s in range(4):
        @pl.when(ngrp > s)
        def _():
          drain_scatter(sems[s])

    def run_table(ids_hbm, tab_hbm, tail_hbm, out_hbm, start, nchunks,
                  total_n, tail_lane0, tail_w):
      pltpu.sync_copy(ids_hbm, ids_v)
      lo = start * _CW
      hi = jnp.where(wid == 31, total_n, (start + nchunks) * _CW)
      cnt = run_filter(lo, hi)

      @pl.when(wid == 31)
      def _():
        pltpu.sync_copy(tail_hbm, bufa_v.at[:, pl.ds(0, tail_w)])
        process_span(bufa_v, tail_lane0, tail_w, cnt, out_hbm)

      bufs = (bufa_v, bufb_v)
      csems = (sma, smb)

      def fire(i, c):
        off = pl.multiple_of((start + c) * _CW, _CW)
        for b in range(4):
          pltpu.async_copy(
              tab_hbm.at[pl.ds(8 * b, 8), pl.ds(off, _CW)],
              bufs[i].at[pl.ds(8 * b, 8), :], csems[i])

      def wait_chunk(i):
        pltpu.make_async_copy(
            tab_hbm.at[:, pl.ds(0, _CW)], bufs[i], csems[i]).wait()

      fire(0, 0)

      def pair(p, carry):
        c0 = 2 * p
        c1 = c0 + 1

        @pl.when(c1 < nchunks)
        def _():
          fire(1, c1)

        wait_chunk(0)
        process_span(bufa_v, (start + c0) * _CW, _CW, cnt, out_hbm)

        @pl.when(c1 < nchunks)
        def _():
          @pl.when(c1 + 1 < nchunks)
          def _():
            fire(0, c1 + 1)

          wait_chunk(1)
          process_span(bufb_v, (start + c1) * _CW, _CW, cnt, out_hbm)

        return carry

      lax.fori_loop(0, lax.div(nchunks + 1, 2), pair, 0)

    u_start = 30 * wid + jnp.minimum(wid, 16)
    u_n = jnp.where(wid < 16, 31, 30)
    run_table(uids_hbm, utab_hbm, utail_hbm, uout_hbm,
              u_start, u_n, _NU, _U_TAIL, _NU - _U_TAIL)
    i_start = 3 * wid + jnp.minimum(wid, 1)
    i_n = jnp.where(wid < 1, 4, 3)
    run_table(iids_hbm, itab_hbm, itail_hbm, iout_hbm,
              i_start, i_n, _NI, _I_TAIL, _NI - _I_TAIL)

  return lookup


def kernel(user_ids, item_ids, user_table, item_table):
  lookup = _make_lookup()
  u128, i128 = lookup(
      user_ids.astype(jnp.int32), item_ids.astype(jnp.int32),
      user_table.T, item_table.T,
      user_table[_U_TAIL:].T, item_table[_I_TAIL:].T)
  return (u128[:, None, :_D], i128[:, None, :_D])


# speedup vs baseline: 1.1404x; 1.1404x over previous
import functools

import jax
import jax.numpy as jnp
from jax import lax
from jax.experimental import pallas as pl
from jax.experimental.pallas import tpu as pltpu
from jax.experimental.pallas import tpu_sc as plsc

_B = 16384
_D = 32
_CW = 1024
_U_MAIN = 999424
_I_MAIN = 99328
_U_TAIL = 999360
_I_TAIL = 99232
_NU = 1000000
_NI = 100000


@functools.lru_cache(maxsize=None)
def _make_lookup():
  info = plsc.get_sparse_core_info()
  NC, NS = info.num_cores, info.num_subcores
  mesh = plsc.VectorSubcoreMesh(core_axis_name="c", subcore_axis_name="s")

  @functools.partial(
      pl.kernel,
      mesh=mesh,
      compiler_params=pltpu.CompilerParams(needs_layout_passes=False),
      out_type=(
          jax.ShapeDtypeStruct((_B, 128), jnp.float32),
          jax.ShapeDtypeStruct((_B, 128), jnp.float32),
      ),
      scratch_types=[
          pltpu.VMEM((_B,), jnp.int32),
          pltpu.VMEM((_B,), jnp.int32),
          pltpu.VMEM((_B,), jnp.int32),
          pltpu.VMEM((_D, _CW), jnp.float32),
          pltpu.VMEM((_D, _CW), jnp.float32),
          pltpu.VMEM((64, 128), jnp.float32),
          pltpu.VMEM((4, 16), jnp.int32),
          pltpu.SemaphoreType.DMA,
          pltpu.SemaphoreType.DMA,
          pltpu.SemaphoreType.DMA,
          pltpu.SemaphoreType.DMA,
          pltpu.SemaphoreType.DMA,
          pltpu.SemaphoreType.DMA,
      ],
  )
  def lookup(uids_hbm, iids_hbm, utab_hbm, itab_hbm, utail_hbm, itail_hbm,
             uout_hbm, iout_hbm,
             ids_v, lst_v, pos_v, bufa_v, bufb_v, outring_v,
             posring_v, sma, smb, sm0, sm1, sm2, sm3):
    wid = lax.axis_index("s") * NC + lax.axis_index("c")
    lanes = lax.iota(jnp.int32, 16)
    sems = [sm0, sm1, sm2, sm3]

    def drain_scatter(sem):
      pltpu.make_async_copy(
          uout_hbm.at[pl.ds(0, 16), :],
          outring_v.at[pl.ds(0, 16), :], sem).wait()

    def run_filter(lo, hi):
      def body(g2, cnt):
        for k in range(2):
          g = g2 * 2 + k
          u = ids_v[pl.ds(g * 16, 16)]
          m = (u >= lo) & (u < hi)
          ranks = plsc.cumsum(jnp.where(m, 1, 0))
          slots = cnt + ranks - 1
          plsc.store_scatter(lst_v, [slots], u, mask=m)
          plsc.store_scatter(pos_v, [slots], lanes + g * 16, mask=m)
          cnt = cnt + plsc.all_reduce_population_count(m)[0]
        return cnt
      return lax.fori_loop(0, _B // 32, body, 0)

    def process_span(buf, lane0, width, cnt, out_hbm):
      def mbody(g, mcnt):
        u = lst_v[pl.ds(g * 16, 16)]
        m = (u >= lane0) & (u < lane0 + width) & (lanes + g * 16 < cnt)
        nm = plsc.all_reduce_population_count(m)[0]

        @pl.when(nm > 0)
        def _():
          ranks = plsc.cumsum(jnp.where(m, 1, 0)) - 1
          plsc.store_scatter(ids_v, [mcnt + ranks], lanes + g * 16, mask=m)

        return mcnt + nm
      nlg = lax.div(cnt + 15, 16)
      mcnt = lax.fori_loop(0, nlg, mbody, 0)
      ngrp = lax.div(mcnt + 15, 16)

      def gbody(g4, carry):
        for s in range(4):
          g = g4 * 4 + s

          @pl.when(g < ngrp)
          def _():
            j = g * 16 + lanes
            jp = jnp.where(j < mcnt, j, jnp.full((16,), 1, jnp.int32) * g * 16)
            idxv = plsc.load_gather(ids_v, [jp])
            uv = plsc.load_gather(lst_v, [idxv])
            pv = plsc.load_gather(pos_v, [idxv])
            ll = uv - lane0

            @pl.when(g >= 4)
            def _():
              drain_scatter(sems[s])

            rows = s * 16 + lanes
            for d in range(_D):
              dv = jnp.full((16,), d, jnp.int32)
              vals = plsc.load_gather(buf, [dv, ll])
              plsc.store_scatter(outring_v, [rows, dv], vals)
            posring_v[s] = pv
            pltpu.async_copy(
                outring_v.at[pl.ds(s * 16, 16), :],
                out_hbm.at[posring_v.at[s]], sems[s])
        return carry

      lax.fori_loop(0, lax.div(ngrp + 3, 4), gbody, 0)
      for s in range(4):
        @pl.when(ngrp > s)
        def _():
          drain_scatter(sems[s])

    def run_table(ids_hbm, tab_hbm, tail_hbm, out_hbm, start, nchunks,
                  total_n, tail_lane0, tail_w):
      pltpu.sync_copy(ids_hbm, ids_v)
      lo = start * _CW
      hi = jnp.where(wid == 31, total_n, (start + nchunks) * _CW)
      cnt = run_filter(lo, hi)

      @pl.when(wid == 31)
      def _():
        pltpu.sync_copy(tail_hbm, bufa_v.at[:, pl.ds(0, tail_w)])
        process_span(bufa_v, tail_lane0, tail_w, cnt, out_hbm)

      bufs = (bufa_v, bufb_v)
      csems = (sma, smb)

      def fire(i, c):
        off = pl.multiple_of((start + c) * _CW, _CW)
        for b in range(4):
          pltpu.async_copy(
              tab_hbm.at[pl.ds(8 * b, 8), pl.ds(off, _CW)],
              bufs[i].at[pl.ds(8 * b, 8), :], csems[i])

      def wait_chunk(i):
        pltpu.make_async_copy(
            tab_hbm.at[:, pl.ds(0, _CW)], bufs[i], csems[i]).wait()

      fire(0, 0)

      def pair(p, carry):
        c0 = 2 * p
        c1 = c0 + 1

        @pl.when(c1 < nchunks)
        def _():
          fire(1, c1)

        wait_chunk(0)
        process_span(bufa_v, (start + c0) * _CW, _CW, cnt, out_hbm)

        @pl.when(c1 < nchunks)
        def _():
          @pl.when(c1 + 1 < nchunks)
          def _():
            fire(0, c1 + 1)

          wait_chunk(1)
          process_span(bufb_v, (start + c1) * _CW, _CW, cnt, out_hbm)

        return carry

      lax.fori_loop(0, lax.div(nchunks + 1, 2), pair, 0)

    u_start = 30 * wid + jnp.minimum(wid, 16)
    u_n = jnp.where(wid < 16, 31, 30)
    run_table(uids_hbm, utab_hbm, utail_hbm, uout_hbm,
              u_start, u_n, _NU, _U_TAIL, _NU - _U_TAIL)
    i_start = 3 * wid + jnp.minimum(wid, 1)
    i_n = jnp.where(wid < 1, 4, 3)
    run_table(iids_hbm, itab_hbm, itail_hbm, iout_hbm,
              i_start, i_n, _NI, _I_TAIL, _NI - _I_TAIL)

  return lookup


def kernel(user_ids, item_ids, user_table, item_table):
  lookup = _make_lookup()
  u128, i128 = lookup(
      user_ids.astype(jnp.int32), item_ids.astype(jnp.int32),
      user_table.T, item_table.T,
      user_table[_U_TAIL:].T, item_table[_I_TAIL:].T)
  return (u128[:, None, :_D], i128[:, None, :_D])


# speedup vs baseline: 1.1751x; 1.0304x over previous
import functools

import jax
import jax.numpy as jnp
from jax import lax
from jax.experimental import pallas as pl
from jax.experimental.pallas import tpu as pltpu
from jax.experimental.pallas import tpu_sc as plsc

_B = 16384
_D = 32
_CW = 1024
_U_MAIN = 999424
_I_MAIN = 99328
_U_TAIL = 999360
_I_TAIL = 99232
_NU = 1000000
_NI = 100000


@functools.lru_cache(maxsize=None)
def _make_lookup():
  info = plsc.get_sparse_core_info()
  NC, NS = info.num_cores, info.num_subcores
  mesh = plsc.VectorSubcoreMesh(core_axis_name="c", subcore_axis_name="s")

  @functools.partial(
      pl.kernel,
      mesh=mesh,
      compiler_params=pltpu.CompilerParams(needs_layout_passes=False),
      out_type=(
          jax.ShapeDtypeStruct((_B, 128), jnp.float32),
          jax.ShapeDtypeStruct((_B, 128), jnp.float32),
      ),
      scratch_types=[
          pltpu.VMEM((_B,), jnp.int32),
          pltpu.VMEM((_B,), jnp.int32),
          pltpu.VMEM((_B,), jnp.int32),
          pltpu.VMEM((_D, _CW), jnp.float32),
          pltpu.VMEM((_D, _CW), jnp.float32),
          pltpu.VMEM((64, 128), jnp.float32),
          pltpu.VMEM((4, 16), jnp.int32),
          pltpu.SemaphoreType.DMA,
          pltpu.SemaphoreType.DMA,
          pltpu.SemaphoreType.DMA,
          pltpu.SemaphoreType.DMA,
          pltpu.SemaphoreType.DMA,
          pltpu.SemaphoreType.DMA,
      ],
  )
  def lookup(uids_hbm, iids_hbm, utab_hbm, itab_hbm, utail_hbm, itail_hbm,
             uout_hbm, iout_hbm,
             ids_v, lst_v, pos_v, bufa_v, bufb_v, outring_v,
             posring_v, sma, smb, sm0, sm1, sm2, sm3):
    wid = lax.axis_index("s") * NC + lax.axis_index("c")
    lanes = lax.iota(jnp.int32, 16)
    sems = [sm0, sm1, sm2, sm3]

    def drain_scatter(sem):
      pltpu.make_async_copy(
          uout_hbm.at[pl.ds(0, 16), :],
          outring_v.at[pl.ds(0, 16), :], sem).wait()

    def run_filter(lo, hi):
      def body(g2, cnt):
        for k in range(2):
          g = g2 * 2 + k
          u = ids_v[pl.ds(g * 16, 16)]
          m = (u >= lo) & (u < hi)
          ranks = plsc.cumsum(jnp.where(m, 1, 0))
          slots = cnt + ranks - 1
          plsc.store_scatter(lst_v, [slots], u, mask=m)
          plsc.store_scatter(pos_v, [slots], lanes + g * 16, mask=m)
          cnt = cnt + plsc.all_reduce_population_count(m)[0]
        return cnt
      return lax.fori_loop(0, _B // 32, body, 0)

    def process_span(buf, lane0, width, cnt, out_hbm):
      def mbody(g2, mcnt):
        for k in range(2):
          g = g2 * 2 + k
          u = lst_v[pl.ds(g * 16, 16)]
          m = (u >= lane0) & (u < lane0 + width) & (lanes + g * 16 < cnt)
          ranks = plsc.cumsum(jnp.where(m, 1, 0))
          plsc.store_scatter(ids_v, [mcnt + ranks - 1], lanes + g * 16,
                             mask=m)
          mcnt = mcnt + plsc.all_reduce_population_count(m)[0]
        return mcnt
      nlg = lax.div(cnt + 31, 32)
      mcnt = lax.fori_loop(0, nlg, mbody, 0)
      ngrp = lax.div(mcnt + 15, 16)

      def gbody(g4, carry):
        for s in range(4):
          g = g4 * 4 + s

          @pl.when(g < ngrp)
          def _():
            j = g * 16 + lanes
            jp = jnp.where(j < mcnt, j, jnp.full((16,), 1, jnp.int32) * g * 16)
            idxv = plsc.load_gather(ids_v, [jp])
            uv = plsc.load_gather(lst_v, [idxv])
            pv = plsc.load_gather(pos_v, [idxv])
            ll = uv - lane0

            @pl.when(g >= 4)
            def _():
              drain_scatter(sems[s])

            rows = s * 16 + lanes
            for d in range(_D):
              dv = jnp.full((16,), d, jnp.int32)
              vals = plsc.load_gather(buf, [dv, ll])
              plsc.store_scatter(outring_v, [rows, dv], vals)
            posring_v[s] = pv
            pltpu.async_copy(
                outring_v.at[pl.ds(s * 16, 16), :],
                out_hbm.at[posring_v.at[s]], sems[s])
        return carry

      lax.fori_loop(0, lax.div(ngrp + 3, 4), gbody, 0)
      for s in range(4):
        @pl.when(ngrp > s)
        def _():
          drain_scatter(sems[s])

    def run_table(ids_hbm, tab_hbm, tail_hbm, out_hbm, start, nchunks,
                  total_n, tail_lane0, tail_w):
      pltpu.sync_copy(ids_hbm, ids_v)
      lo = start * _CW
      hi = jnp.where(wid == 31, total_n, (start + nchunks) * _CW)
      cnt = run_filter(lo, hi)

      @pl.when(wid == 31)
      def _():
        pltpu.sync_copy(tail_hbm, bufa_v.at[:, pl.ds(0, tail_w)])
        process_span(bufa_v, tail_lane0, tail_w, cnt, out_hbm)

      bufs = (bufa_v, bufb_v)
      csems = (sma, smb)

      def fire(i, c):
        off = pl.multiple_of((start + c) * _CW, _CW)
        for b in range(4):
          pltpu.async_copy(
              tab_hbm.at[pl.ds(8 * b, 8), pl.ds(off, _CW)],
              bufs[i].at[pl.ds(8 * b, 8), :], csems[i])

      def wait_chunk(i):
        pltpu.make_async_copy(
            tab_hbm.at[:, pl.ds(0, _CW)], bufs[i], csems[i]).wait()

      fire(0, 0)

      def pair(p, carry):
        c0 = 2 * p
        c1 = c0 + 1

        @pl.when(c1 < nchunks)
        def _():
          fire(1, c1)

        wait_chunk(0)
        process_span(bufa_v, (start + c0) * _CW, _CW, cnt, out_hbm)

        @pl.when(c1 < nchunks)
        def _():
          @pl.when(c1 + 1 < nchunks)
          def _():
            fire(0, c1 + 1)

          wait_chunk(1)
          process_span(bufb_v, (start + c1) * _CW, _CW, cnt, out_hbm)

        return carry

      lax.fori_loop(0, lax.div(nchunks + 1, 2), pair, 0)

    u_start = 30 * wid + jnp.minimum(wid, 16)
    u_n = jnp.where(wid < 16, 31, 30)
    run_table(uids_hbm, utab_hbm, utail_hbm, uout_hbm,
              u_start, u_n, _NU, _U_TAIL, _NU - _U_TAIL)
    i_start = 3 * wid + jnp.minimum(wid, 1)
    i_n = jnp.where(wid < 1, 4, 3)
    run_table(iids_hbm, itab_hbm, itail_hbm, iout_hbm,
              i_start, i_n, _NI, _I_TAIL, _NI - _I_TAIL)

  return lookup


def kernel(user_ids, item_ids, user_table, item_table):
  lookup = _make_lookup()
  u128, i128 = lookup(
      user_ids.astype(jnp.int32), item_ids.astype(jnp.int32),
      user_table.T, item_table.T,
      user_table[_U_TAIL:].T, item_table[_I_TAIL:].T)
  return (u128[:, None, :_D], i128[:, None, :_D])


# speedup vs baseline: 1.2945x; 1.1016x over previous
import functools

import jax
import jax.numpy as jnp
from jax import lax
from jax.experimental import pallas as pl
from jax.experimental.pallas import tpu as pltpu
from jax.experimental.pallas import tpu_sc as plsc

_B = 16384
_D = 32
_CW = 1024
_U_MAIN = 999424
_I_MAIN = 99328
_U_TAIL = 999360
_I_TAIL = 99232
_NU = 1000000
_NI = 100000


@functools.lru_cache(maxsize=None)
def _make_lookup():
  info = plsc.get_sparse_core_info()
  NC, NS = info.num_cores, info.num_subcores
  mesh = plsc.VectorSubcoreMesh(core_axis_name="c", subcore_axis_name="s")

  @functools.partial(
      pl.kernel,
      mesh=mesh,
      compiler_params=pltpu.CompilerParams(needs_layout_passes=False),
      out_type=(
          jax.ShapeDtypeStruct((_B, 128), jnp.float32),
          jax.ShapeDtypeStruct((_B, 128), jnp.float32),
      ),
      scratch_types=[
          pltpu.VMEM((_B,), jnp.int32),
          pltpu.VMEM((_B,), jnp.int32),
          pltpu.VMEM((_B,), jnp.int32),
          pltpu.VMEM((_D, _CW), jnp.float32),
          pltpu.VMEM((_D, _CW), jnp.float32),
          pltpu.VMEM((64, 128), jnp.float32),
          pltpu.VMEM((4, 16), jnp.int32),
          pltpu.SemaphoreType.DMA,
          pltpu.SemaphoreType.DMA,
          pltpu.SemaphoreType.DMA,
          pltpu.SemaphoreType.DMA,
          pltpu.SemaphoreType.DMA,
          pltpu.SemaphoreType.DMA,
      ],
  )
  def lookup(uids_hbm, iids_hbm, utab_hbm, itab_hbm, utail_hbm, itail_hbm,
             uout_hbm, iout_hbm,
             ids_v, lst_v, pos_v, bufa_v, bufb_v, outring_v,
             posring_v, sma, smb, sm0, sm1, sm2, sm3):
    wid = lax.axis_index("s") * NC + lax.axis_index("c")
    lanes = lax.iota(jnp.int32, 16)
    sems = [sm0, sm1, sm2, sm3]

    def drain_scatter(sem):
      pltpu.make_async_copy(
          uout_hbm.at[pl.ds(0, 16), :],
          outring_v.at[pl.ds(0, 16), :], sem).wait()

    def run_filter(lo, hi):
      def body(g4, cnt):
        us, ms, ranks = [], [], []
        for k in range(4):
          g = g4 * 4 + k
          u = ids_v[pl.ds(g * 16, 16)]
          m = (u >= lo) & (u < hi)
          us.append(u)
          ms.append(m)
          ranks.append(plsc.cumsum(jnp.where(m, 1, 0)))
        for k in range(4):
          g = g4 * 4 + k
          slots = cnt + ranks[k] - 1
          plsc.store_scatter(lst_v, [slots], us[k], mask=ms[k])
          plsc.store_scatter(pos_v, [slots], lanes + g * 16, mask=ms[k])
          cnt = cnt + plsc.all_reduce_population_count(ms[k])[0]
        return cnt
      return lax.fori_loop(0, _B // 64, body, 0)

    def process_span(buf, lane0, width, cnt, out_hbm):
      def mbody(g2, mcnt):
        for k in range(2):
          g = g2 * 2 + k
          u = lst_v[pl.ds(g * 16, 16)]
          m = (u >= lane0) & (u < lane0 + width) & (lanes + g * 16 < cnt)
          ranks = plsc.cumsum(jnp.where(m, 1, 0))
          plsc.store_scatter(ids_v, [mcnt + ranks - 1], lanes + g * 16,
                             mask=m)
          mcnt = mcnt + plsc.all_reduce_population_count(m)[0]
        return mcnt
      nlg = lax.div(cnt + 31, 32)
      mcnt = lax.fori_loop(0, nlg, mbody, 0)
      ngrp = lax.div(mcnt + 15, 16)

      def gbody(g4, carry):
        for s in range(4):
          g = g4 * 4 + s

          @pl.when(g < ngrp)
          def _():
            j = g * 16 + lanes
            jp = jnp.where(j < mcnt, j, jnp.full((16,), 1, jnp.int32) * g * 16)
            idxv = plsc.load_gather(ids_v, [jp])
            uv = plsc.load_gather(lst_v, [idxv])
            pv = plsc.load_gather(pos_v, [idxv])
            ll = uv - lane0

            @pl.when(g >= 4)
            def _():
              drain_scatter(sems[s])

            rows = s * 16 + lanes
            for d in range(_D):
              dv = jnp.full((16,), d, jnp.int32)
              vals = plsc.load_gather(buf, [dv, ll])
              plsc.store_scatter(outring_v, [rows, dv], vals)
            posring_v[s] = pv
            pltpu.async_copy(
                outring_v.at[pl.ds(s * 16, 16), :],
                out_hbm.at[posring_v.at[s]], sems[s])
        return carry

      lax.fori_loop(0, lax.div(ngrp + 3, 4), gbody, 0)
      for s in range(4):
        @pl.when(ngrp > s)
        def _():
          drain_scatter(sems[s])

    def run_table(ids_hbm, tab_hbm, tail_hbm, out_hbm, start, nchunks,
                  total_n, tail_lane0, tail_w):
      pltpu.sync_copy(ids_hbm, ids_v)
      lo = start * _CW
      hi = jnp.where(wid == 31, total_n, (start + nchunks) * _CW)
      cnt = run_filter(lo, hi)

      @pl.when(wid == 31)
      def _():
        pltpu.sync_copy(tail_hbm, bufa_v.at[:, pl.ds(0, tail_w)])
        process_span(bufa_v, tail_lane0, tail_w, cnt, out_hbm)

      bufs = (bufa_v, bufb_v)
      csems = (sma, smb)

      def fire(i, c):
        off = pl.multiple_of((start + c) * _CW, _CW)
        for b in range(4):
          pltpu.async_copy(
              tab_hbm.at[pl.ds(8 * b, 8), pl.ds(off, _CW)],
              bufs[i].at[pl.ds(8 * b, 8), :], csems[i])

      def wait_chunk(i):
        pltpu.make_async_copy(
            tab_hbm.at[:, pl.ds(0, _CW)], bufs[i], csems[i]).wait()

      fire(0, 0)

      def pair(p, carry):
        c0 = 2 * p
        c1 = c0 + 1

        @pl.when(c1 < nchunks)
        def _():
          fire(1, c1)

        wait_chunk(0)
        process_span(bufa_v, (start + c0) * _CW, _CW, cnt, out_hbm)

        @pl.when(c1 < nchunks)
        def _():
          @pl.when(c1 + 1 < nchunks)
          def _():
            fire(0, c1 + 1)

          wait_chunk(1)
          process_span(bufb_v, (start + c1) * _CW, _CW, cnt, out_hbm)

        return carry

      lax.fori_loop(0, lax.div(nchunks + 1, 2), pair, 0)

    u_start = 30 * wid + jnp.minimum(wid, 16)
    u_n = jnp.where(wid < 16, 31, 30)
    run_table(uids_hbm, utab_hbm, utail_hbm, uout_hbm,
              u_start, u_n, _NU, _U_TAIL, _NU - _U_TAIL)
    i_start = 3 * wid + jnp.minimum(wid, 1)
    i_n = jnp.where(wid < 1, 4, 3)
    run_table(iids_hbm, itab_hbm, itail_hbm, iout_hbm,
              i_start, i_n, _NI, _I_TAIL, _NI - _I_TAIL)

  return lookup


def kernel(user_ids, item_ids, user_table, item_table):
  lookup = _make_lookup()
  u128, i128 = lookup(
      user_ids.astype(jnp.int32), item_ids.astype(jnp.int32),
      user_table.T, item_table.T,
      user_table[_U_TAIL:].T, item_table[_I_TAIL:].T)
  return (u128[:, None, :_D], i128[:, None, :_D])


# speedup vs baseline: 1.3107x; 1.0125x over previous
import functools

import jax
import jax.numpy as jnp
from jax import lax
from jax.experimental import pallas as pl
from jax.experimental.pallas import tpu as pltpu
from jax.experimental.pallas import tpu_sc as plsc

_B = 16384
_D = 32
_CW = 1024
_U_MAIN = 999424
_I_MAIN = 99328
_U_TAIL = 999360
_I_TAIL = 99232
_NU = 1000000
_NI = 100000


@functools.lru_cache(maxsize=None)
def _make_lookup():
  info = plsc.get_sparse_core_info()
  NC, NS = info.num_cores, info.num_subcores
  mesh = plsc.VectorSubcoreMesh(core_axis_name="c", subcore_axis_name="s")

  @functools.partial(
      pl.kernel,
      mesh=mesh,
      compiler_params=pltpu.CompilerParams(needs_layout_passes=False),
      out_type=(
          jax.ShapeDtypeStruct((_B, 128), jnp.float32),
          jax.ShapeDtypeStruct((_B, 128), jnp.float32),
      ),
      scratch_types=[
          pltpu.VMEM((_B,), jnp.int32),
          pltpu.VMEM((_B,), jnp.int32),
          pltpu.VMEM((_B,), jnp.int32),
          pltpu.VMEM((_D, _CW), jnp.float32),
          pltpu.VMEM((_D, _CW), jnp.float32),
          pltpu.VMEM((64, 128), jnp.float32),
          pltpu.VMEM((4, 16), jnp.int32),
          pltpu.SemaphoreType.DMA,
          pltpu.SemaphoreType.DMA,
          pltpu.SemaphoreType.DMA,
          pltpu.SemaphoreType.DMA,
          pltpu.SemaphoreType.DMA,
          pltpu.SemaphoreType.DMA,
      ],
  )
  def lookup(uids_hbm, iids_hbm, utab_hbm, itab_hbm, utail_hbm, itail_hbm,
             uout_hbm, iout_hbm,
             ids_v, lst_v, pos_v, bufa_v, bufb_v, outring_v,
             posring_v, sma, smb, sm0, sm1, sm2, sm3):
    wid = lax.axis_index("s") * NC + lax.axis_index("c")
    lanes = lax.iota(jnp.int32, 16)
    sems = [sm0, sm1, sm2, sm3]

    def drain_scatter(sem):
      pltpu.make_async_copy(
          uout_hbm.at[pl.ds(0, 16), :],
          outring_v.at[pl.ds(0, 16), :], sem).wait()

    def run_filter(lo, hi):
      def body(g4, cnt):
        us, ms, ranks = [], [], []
        for k in range(4):
          g = g4 * 4 + k
          u = ids_v[pl.ds(g * 16, 16)]
          m = (u >= lo) & (u < hi)
          us.append(u)
          ms.append(m)
          ranks.append(plsc.cumsum(jnp.where(m, 1, 0)))
        for k in range(4):
          g = g4 * 4 + k
          slots = cnt + ranks[k] - 1
          plsc.store_scatter(lst_v, [slots], us[k], mask=ms[k])
          plsc.store_scatter(pos_v, [slots], lanes + g * 16, mask=ms[k])
          cnt = cnt + plsc.all_reduce_population_count(ms[k])[0]
        return cnt
      return lax.fori_loop(0, _B // 64, body, 0)

    def process_span(buf, lane0, width, cnt, out_hbm):
      def mbody(g4, mcnt):
        ms, ranks = [], []
        for k in range(4):
          g = g4 * 4 + k
          u = lst_v[pl.ds(g * 16, 16)]
          m = (u >= lane0) & (u < lane0 + width) & (lanes + g * 16 < cnt)
          ms.append(m)
          ranks.append(plsc.cumsum(jnp.where(m, 1, 0)))
        for k in range(4):
          g = g4 * 4 + k
          plsc.store_scatter(ids_v, [mcnt + ranks[k] - 1], lanes + g * 16,
                             mask=ms[k])
          mcnt = mcnt + plsc.all_reduce_population_count(ms[k])[0]
        return mcnt
      nlg = lax.div(cnt + 63, 64)
      mcnt = lax.fori_loop(0, nlg, mbody, 0)
      ngrp = lax.div(mcnt + 15, 16)

      def gbody(g4, carry):
        for s in range(4):
          g = g4 * 4 + s

          @pl.when(g < ngrp)
          def _():
            j = g * 16 + lanes
            jp = jnp.where(j < mcnt, j, jnp.full((16,), 1, jnp.int32) * g * 16)
            idxv = plsc.load_gather(ids_v, [jp])
            uv = plsc.load_gather(lst_v, [idxv])
            pv = plsc.load_gather(pos_v, [idxv])
            ll = uv - lane0

            @pl.when(g >= 4)
            def _():
              drain_scatter(sems[s])

            rows = s * 16 + lanes
            for d in range(_D):
              dv = jnp.full((16,), d, jnp.int32)
              vals = plsc.load_gather(buf, [dv, ll])
              plsc.store_scatter(outring_v, [rows, dv], vals)
            posring_v[s] = pv
            pltpu.async_copy(
                outring_v.at[pl.ds(s * 16, 16), :],
                out_hbm.at[posring_v.at[s]], sems[s])
        return carry

      lax.fori_loop(0, lax.div(ngrp + 3, 4), gbody, 0)
      for s in range(4):
        @pl.when(ngrp > s)
        def _():
          drain_scatter(sems[s])

    def run_table(ids_hbm, tab_hbm, tail_hbm, out_hbm, start, nchunks,
                  total_n, tail_lane0, tail_w):
      pltpu.sync_copy(ids_hbm, ids_v)
      lo = start * _CW
      hi = jnp.where(wid == 31, total_n, (start + nchunks) * _CW)
      cnt = run_filter(lo, hi)

      @pl.when(wid == 31)
      def _():
        pltpu.sync_copy(tail_hbm, bufa_v.at[:, pl.ds(0, tail_w)])
        process_span(bufa_v, tail_lane0, tail_w, cnt, out_hbm)

      bufs = (bufa_v, bufb_v)
      csems = (sma, smb)

      def fire(i, c):
        off = pl.multiple_of((start + c) * _CW, _CW)
        for b in range(4):
          pltpu.async_copy(
              tab_hbm.at[pl.ds(8 * b, 8), pl.ds(off, _CW)],
              bufs[i].at[pl.ds(8 * b, 8), :], csems[i])

      def wait_chunk(i):
        pltpu.make_async_copy(
            tab_hbm.at[:, pl.ds(0, _CW)], bufs[i], csems[i]).wait()

      fire(0, 0)

      def pair(p, carry):
        c0 = 2 * p
        c1 = c0 + 1

        @pl.when(c1 < nchunks)
        def _():
          fire(1, c1)

        wait_chunk(0)
        process_span(bufa_v, (start + c0) * _CW, _CW, cnt, out_hbm)

        @pl.when(c1 < nchunks)
        def _():
          @pl.when(c1 + 1 < nchunks)
          def _():
            fire(0, c1 + 1)

          wait_chunk(1)
          process_span(bufb_v, (start + c1) * _CW, _CW, cnt, out_hbm)

        return carry

      lax.fori_loop(0, lax.div(nchunks + 1, 2), pair, 0)

    u_start = 30 * wid + jnp.minimum(wid, 16)
    u_n = jnp.where(wid < 16, 31, 30)
    run_table(uids_hbm, utab_hbm, utail_hbm, uout_hbm,
              u_start, u_n, _NU, _U_TAIL, _NU - _U_TAIL)
    i_start = 3 * wid + jnp.minimum(wid, 1)
    i_n = jnp.where(wid < 1, 4, 3)
    run_table(iids_hbm, itab_hbm, itail_hbm, iout_hbm,
              i_start, i_n, _NI, _I_TAIL, _NI - _I_TAIL)

  return lookup


def kernel(user_ids, item_ids, user_table, item_table):
  lookup = _make_lookup()
  u128, i128 = lookup(
      user_ids.astype(jnp.int32), item_ids.astype(jnp.int32),
      user_table.T, item_table.T,
      user_table[_U_TAIL:].T, item_table[_I_TAIL:].T)
  return (u128[:, None, :_D], i128[:, None, :_D])
